# baseline (speedup 1.0000x reference)
import functools

import jax
import jax.numpy as jnp
from jax import lax
from jax.experimental import pallas as pl
from jax.experimental.pallas import tpu as pltpu
from jax.experimental.pallas import tpu_sc as plsc

_NC = 2
_NS = 16
_NW = _NC * _NS
_L = 16
_CHUNK = 128
_INFLIGHT = 32


@functools.lru_cache(maxsize=None)
def _lr_call(B, F, V):
    n_per_w = (B * F) // _NW
    rows_per_w = n_per_w // _CHUNK
    b_per_w = B // _NW
    groups = b_per_w // _L
    cpw = b_per_w // _CHUNK

    mesh = plsc.VectorSubcoreMesh(core_axis_name="c", subcore_axis_name="s")

    @functools.partial(
        pl.kernel,
        out_type=jax.ShapeDtypeStruct((B,), jnp.float32),
        mesh=mesh,
        scratch_types=[
            pltpu.VMEM((F, b_per_w), jnp.int32),
            pltpu.VMEM((1, n_per_w), jnp.float32),
            pltpu.VMEM((b_per_w,), jnp.float32),
            pltpu.SemaphoreType.DMA,
            pltpu.SemaphoreType.DMA,
            pltpu.SemaphoreType.DMA,
            pltpu.SemaphoreType.DMA,
        ],
    )
    def lr_kernel(xt_hbm, table_hbm, out_hbm, idx_v, vals_v, out_v,
                  sem_a, sem_b, sem_c, sem_d):
        wid = lax.axis_index("s") * _NC + lax.axis_index("c")
        base = wid * b_per_w
        sems = (sem_a, sem_b, sem_c, sem_d)
        chunk_bytes_cols = F * _CHUNK

        def stage(c):
            pltpu.sync_copy(
                xt_hbm.at[:, pl.ds(base + c * _CHUNK, _CHUNK)],
                idx_v.at[:, pl.ds(c * _CHUNK, _CHUNK)],
            )

        def fire_chunk(c, sem):
            def fk(k, _):
                pltpu.async_copy(
                    table_hbm.at[idx_v.at[pl.ds(k, 1),
                                          pl.ds(c * _CHUNK, _CHUNK)]],
                    vals_v.at[:, pl.ds(k * b_per_w + c * _CHUNK, _CHUNK)],
                    sem,
                )
                return 0
            lax.fori_loop(0, F, fk, 0, unroll=2)

        def drain(sem):
            pltpu.make_async_copy(
                table_hbm.at[:, pl.ds(0, chunk_bytes_cols)],
                vals_v.at[:, pl.ds(0, chunk_bytes_cols)],
                sem,
            ).wait()

        def compute(c):
            def seg(gl, _):
                g = c * (_CHUNK // _L) + gl
                acc = jnp.zeros((_L,), jnp.float32)
                for k in range(F):
                    acc = acc + vals_v[0, pl.ds(k * b_per_w + g * _L, _L)]
                out_v[pl.ds(g * _L, _L)] = acc
                return 0
            lax.fori_loop(0, _CHUNK // _L, seg, 0, unroll=2)

        for c in range(cpw):
            stage(c)
            fire_chunk(c, sems[c])
        for c in range(cpw):
            drain(sems[c])
            compute(c)

        pltpu.sync_copy(out_v, out_hbm.at[pl.ds(base, b_per_w)])

    return lr_kernel


@functools.lru_cache(maxsize=None)
def _fm_call(B, F, D):
    BB = 2048

    def _fm_body(x_ref, o_ref):
        x = x_ref[...]
        x3 = x.reshape(F, D, BB)
        s = jnp.sum(x3, axis=0)
        ss = jnp.sum(s * s, axis=0)
        sq = jnp.sum(x * x, axis=0)
        o_ref[...] = 0.5 * (ss - sq)

    return pl.pallas_call(
        _fm_body,
        grid=(B // BB,),
        in_specs=[pl.BlockSpec((F * D, BB), lambda i: (0, i))],
        out_specs=pl.BlockSpec((BB,), lambda i: (i,)),
        out_shape=jax.ShapeDtypeStruct((B,), jnp.float32),
    )


def kernel(X, feature_emb, lr_table, bias):
    B, F = X.shape
    D = feature_emb.shape[2]
    V = lr_table.shape[0]

    lr = _lr_call(B, F, V)(X.T, lr_table.T)

    xT = feature_emb.reshape(B, F * D).T
    fm = _fm_call(B, F, D)(xT)

    return (fm + lr + bias[0])[:, None]


# speedup vs baseline: 1.0107x; 1.0107x over previous
import functools

import jax
import jax.numpy as jnp
from jax import lax
from jax.experimental import pallas as pl
from jax.experimental.pallas import tpu as pltpu
from jax.experimental.pallas import tpu_sc as plsc

_NC = 2
_NS = 16
_NW = _NC * _NS
_L = 16
_CHUNK = 128


@functools.lru_cache(maxsize=None)
def _lr_call(B, F, V):
    n_per_w = (B * F) // _NW
    b_per_w = B // _NW
    cpw = b_per_w // _CHUNK

    mesh = plsc.VectorSubcoreMesh(core_axis_name="c", subcore_axis_name="s")

    @functools.partial(
        pl.kernel,
        out_type=jax.ShapeDtypeStruct((B,), jnp.float32),
        mesh=mesh,
        scratch_types=[
            pltpu.VMEM((F, b_per_w), jnp.int32),
            pltpu.VMEM((1, n_per_w), jnp.float32),
            pltpu.VMEM((b_per_w,), jnp.float32),
            pltpu.SemaphoreType.DMA,
            pltpu.SemaphoreType.DMA,
        ],
    )
    def lr_kernel(xt_hbm, table_hbm, out_hbm, idx_v, vals_v, out_v,
                  sem_a, sem_b):
        wid = lax.axis_index("s") * _NC + lax.axis_index("c")
        base = wid * b_per_w
        sems = (sem_a, sem_b)
        chunk_bytes_cols = F * _CHUNK

        def stage(c):
            pltpu.sync_copy(
                xt_hbm.at[:, pl.ds(base + c * _CHUNK, _CHUNK)],
                idx_v.at[:, pl.ds(c * _CHUNK, _CHUNK)],
            )

        def fire_chunk(c, sem):
            def fk(k, _):
                pltpu.async_copy(
                    table_hbm.at[idx_v.at[pl.ds(k, 1),
                                          pl.ds(c * _CHUNK, _CHUNK)]],
                    vals_v.at[:, pl.ds(k * b_per_w + c * _CHUNK, _CHUNK)],
                    sem,
                )
                return 0
            lax.fori_loop(0, F, fk, 0, unroll=2)

        def drain(sem):
            pltpu.make_async_copy(
                table_hbm.at[:, pl.ds(0, chunk_bytes_cols)],
                vals_v.at[:, pl.ds(0, chunk_bytes_cols)],
                sem,
            ).wait()

        def compute(c):
            def seg(gl, _):
                g = c * (_CHUNK // _L) + gl
                acc = jnp.zeros((_L,), jnp.float32)
                for k in range(F):
                    acc = acc + vals_v[0, pl.ds(k * b_per_w + g * _L, _L)]
                out_v[pl.ds(g * _L, _L)] = acc
                return 0
            lax.fori_loop(0, _CHUNK // _L, seg, 0, unroll=2)

        stage(0)
        fire_chunk(0, sems[0])
        stage(1)
        fire_chunk(1, sems[1])

        def step(c, _):
            def on_parity(par):
                def _():
                    sem = sems[par]
                    drain(sem)

                    @pl.when(c + 2 < cpw)
                    def _():
                        stage(c + 2)
                        fire_chunk(c + 2, sem)
                return _

            pl.when(lax.rem(c, 2) == 0)(on_parity(0))
            pl.when(lax.rem(c, 2) == 1)(on_parity(1))
            compute(c)
            return 0

        lax.fori_loop(0, cpw, step, 0)

        pltpu.sync_copy(out_v, out_hbm.at[pl.ds(base, b_per_w)])

    return lr_kernel


@functools.lru_cache(maxsize=None)
def _fm_call(B, F, D):
    BB = 2048

    def _fm_body(x_ref, o_ref):
        x = x_ref[...]
        x3 = x.reshape(F, D, BB)
        s = jnp.sum(x3, axis=0)
        ss = jnp.sum(s * s, axis=0)
        sq = jnp.sum(x * x, axis=0)
        o_ref[...] = 0.5 * (ss - sq)

    return pl.pallas_call(
        _fm_body,
        grid=(B // BB,),
        in_specs=[pl.BlockSpec((F * D, BB), lambda i: (0, i))],
        out_specs=pl.BlockSpec((BB,), lambda i: (i,)),
        out_shape=jax.ShapeDtypeStruct((B,), jnp.float32),
    )


def kernel(X, feature_emb, lr_table, bias):
    B, F = X.shape
    D = feature_emb.shape[2]
    V = lr_table.shape[0]

    lr = _lr_call(B, F, V)(X.T, lr_table.T)

    xT = feature_emb.reshape(B, F * D).T
    fm = _fm_call(B, F, D)(xT)

    return (fm + lr + bias[0])[:, None]


# speedup vs baseline: 1.0166x; 1.0059x over previous
import functools

import jax
import jax.numpy as jnp
from jax import lax
from jax.experimental import pallas as pl
from jax.experimental.pallas import tpu as pltpu
from jax.experimental.pallas import tpu_sc as plsc

_NC = 2
_NS = 16
_NW = _NC * _NS
_L = 16
_CHUNK = 128


@functools.lru_cache(maxsize=None)
def _lr_call(B, F, V):
    n_per_w = (B * F) // _NW
    b_per_w = B // _NW
    cpw = b_per_w // _CHUNK

    mesh = plsc.VectorSubcoreMesh(core_axis_name="c", subcore_axis_name="s")

    @functools.partial(
        pl.kernel,
        out_type=jax.ShapeDtypeStruct((B,), jnp.float32),
        mesh=mesh,
        scratch_types=[
            pltpu.VMEM((F, b_per_w), jnp.int32),
            pltpu.VMEM((1, n_per_w), jnp.float32),
            pltpu.VMEM((b_per_w,), jnp.float32),
            pltpu.SemaphoreType.DMA,
            pltpu.SemaphoreType.DMA,
            pltpu.SemaphoreType.DMA,
            pltpu.SemaphoreType.DMA,
        ],
    )
    def lr_kernel(xt_hbm, table_hbm, out_hbm, idx_v, vals_v, out_v,
                  sem_a, sem_b, sem_s, sem_o):
        wid = lax.axis_index("s") * _NC + lax.axis_index("c")
        base = wid * b_per_w
        sems = (sem_a, sem_b)
        chunk_bytes_cols = F * _CHUNK

        def stage(c):
            pltpu.async_copy(
                xt_hbm.at[:, pl.ds(base + c * _CHUNK, _CHUNK)],
                idx_v.at[:, pl.ds(c * _CHUNK, _CHUNK)],
                sem_s,
            )

        def stage_wait():
            pltpu.make_async_copy(
                xt_hbm.at[:, pl.ds(base, _CHUNK)],
                idx_v.at[:, pl.ds(0, _CHUNK)],
                sem_s,
            ).wait()

        def fire_chunk(c, sem):
            def fk(k, _):
                pltpu.async_copy(
                    table_hbm.at[idx_v.at[pl.ds(k, 1),
                                          pl.ds(c * _CHUNK, _CHUNK)]],
                    vals_v.at[:, pl.ds(k * b_per_w + c * _CHUNK, _CHUNK)],
                    sem,
                )
                return 0
            lax.fori_loop(0, F, fk, 0, unroll=2)

        def drain(sem):
            pltpu.make_async_copy(
                table_hbm.at[:, pl.ds(0, chunk_bytes_cols)],
                vals_v.at[:, pl.ds(0, chunk_bytes_cols)],
                sem,
            ).wait()

        def compute(c):
            def seg(gl, _):
                g = c * (_CHUNK // _L) + gl
                acc = jnp.zeros((_L,), jnp.float32)
                for k in range(F):
                    acc = acc + vals_v[0, pl.ds(k * b_per_w + g * _L, _L)]
                out_v[pl.ds(g * _L, _L)] = acc
                return 0
            lax.fori_loop(0, _CHUNK // _L, seg, 0, unroll=2)

        stage(0)
        stage(1)
        stage_wait()
        fire_chunk(0, sems[0])
        stage_wait()
        fire_chunk(1, sems[1])

        def step(c, _):
            @pl.when(c + 2 < cpw)
            def _():
                stage(c + 2)

            def on_parity(par):
                def _():
                    sem = sems[par]
                    drain(sem)

                    @pl.when(c + 2 < cpw)
                    def _():
                        stage_wait()
                        fire_chunk(c + 2, sem)
                return _

            pl.when(lax.rem(c, 2) == 0)(on_parity(0))
            pl.when(lax.rem(c, 2) == 1)(on_parity(1))
            compute(c)
            pltpu.async_copy(
                out_v.at[pl.ds(c * _CHUNK, _CHUNK)],
                out_hbm.at[pl.ds(base + c * _CHUNK, _CHUNK)],
                sem_o,
            )
            return 0

        lax.fori_loop(0, cpw, step, 0)

        pltpu.make_async_copy(out_v, out_hbm.at[pl.ds(base, b_per_w)],
                              sem_o).wait()

    return lr_kernel


@functools.lru_cache(maxsize=None)
def _fm_call(B, F, D):
    BB = 2048

    def _fm_body(x_ref, o_ref):
        x = x_ref[...]
        x3 = x.reshape(F, D, BB)
        s = jnp.sum(x3, axis=0)
        ss = jnp.sum(s * s, axis=0)
        sq = jnp.sum(x * x, axis=0)
        o_ref[...] = 0.5 * (ss - sq)

    return pl.pallas_call(
        _fm_body,
        grid=(B // BB,),
        in_specs=[pl.BlockSpec((F * D, BB), lambda i: (0, i))],
        out_specs=pl.BlockSpec((BB,), lambda i: (i,)),
        out_shape=jax.ShapeDtypeStruct((B,), jnp.float32),
    )


def kernel(X, feature_emb, lr_table, bias):
    B, F = X.shape
    D = feature_emb.shape[2]
    V = lr_table.shape[0]

    lr = _lr_call(B, F, V)(X.T, lr_table.T)

    xT = feature_emb.reshape(B, F * D).T
    fm = _fm_call(B, F, D)(xT)

    return (fm + lr + bias[0])[:, None]
